# baseline (speedup 1.0000x reference)
import functools

import jax
import jax.numpy as jnp
from jax import lax
from jax.experimental import pallas as pl
from jax.experimental.pallas import tpu as pltpu
from jax.experimental.pallas import tpu_sc as plsc

H = 128
NEI = 8
NC, NS = 2, 16
LOG2E = 1.4426950408889634
NW = NC * NS
LANES = 16
BT = 512


def _row_mask0(x, pid, nrows):
    rows = pid * nrows + lax.broadcasted_iota(jnp.int32, x.shape, 0)
    return jnp.where(rows == 0, 0.0, x)


def _pre_body(x_ref, wzx_ref, bz_ref, whx_ref, bh_ref, wr_ref, br_ref, ur_ref,
              xwz_ref, xwh_ref, r1nb_ref, t_ref):
    pid = pl.program_id(0)
    x = x_ref[...]
    xwz = jnp.dot(x, wzx_ref[...], preferred_element_type=jnp.float32) + bz_ref[...]
    xwh = jnp.dot(x, whx_ref[...], preferred_element_type=jnp.float32) + bh_ref[...]
    r1nb = -(jnp.dot(x, wr_ref[...], preferred_element_type=jnp.float32) + br_ref[...])
    xwz_ref[...] = xwz
    xwh_ref[...] = xwh
    r1nb_ref[...] = r1nb
    h1 = jax.nn.sigmoid(xwz) * jnp.tanh(xwh)
    h1 = _row_mask0(h1, pid, x.shape[0])
    t_ref[:, :H] = h1
    t_ref[:, H:] = jnp.dot(h1, ur_ref[...], preferred_element_type=jnp.float32)


def _upd_body(s_ref, xwz_ref, xwh_ref, wzh_ref, whh_ref, ur_ref, t_ref):
    pid = pl.program_id(0)
    sum_h = s_ref[:, :H]
    sum_g = s_ref[:, H:]
    z = jax.nn.sigmoid(
        xwz_ref[...] + jnp.dot(sum_h, wzh_ref[...], preferred_element_type=jnp.float32))
    pre = jnp.tanh(
        xwh_ref[...] + jnp.dot(sum_g, whh_ref[...], preferred_element_type=jnp.float32))
    nh = (1.0 - z) * sum_h + z * pre
    nh = _row_mask0(nh, pid, sum_h.shape[0])
    t_ref[:, :H] = nh
    t_ref[:, H:] = jnp.dot(nh, ur_ref[...], preferred_element_type=jnp.float32)


def _make_sc_gather(E, C):
    EPW = E // NW
    NCHUNK = EPW // C
    mesh = plsc.VectorSubcoreMesh(core_axis_name="c", subcore_axis_name="s",
                                  num_cores=NC, num_subcores=NS)

    assert NCHUNK % 2 == 0

    @functools.partial(
        pl.kernel,
        out_type=jax.ShapeDtypeStruct((E, 2 * H), jnp.float32),
        mesh=mesh,
        scratch_types=[
            [pltpu.VMEM((C * NEI,), jnp.int32)] * 2,
            [pltpu.VMEM((C * NEI, 2 * H), jnp.float32)] * 2,
            [pltpu.VMEM((C, H), jnp.float32)] * 2,
            [pltpu.VMEM((C, 2 * H), jnp.float32)] * 2,
            [pltpu.SemaphoreType.DMA] * 2,
            [pltpu.SemaphoreType.DMA] * 2,
            [pltpu.SemaphoreType.DMA] * 2,
            [pltpu.SemaphoreType.DMA] * 2,
        ],
    )
    def sc_gather(t_hbm, idx_hbm, r1nb_hbm, s_hbm,
                  idx_v, rows_v, r1_v, out_v, sem_i, sem_g, sem_r, sem_o):
        wid = lax.axis_index("s") * NC + lax.axis_index("c")
        base_w = wid * EPW

        def ebase(c):
            return pl.multiple_of(base_w + c * C, 8)

        def ibase(c):
            return pl.multiple_of((base_w + c * C) * NEI, 8)

        def idx_copy(c, b):
            return pltpu.make_async_copy(
                idx_hbm.at[pl.ds(ibase(c), C * NEI)], idx_v[b], sem_i[b])

        def row_copy(b):
            return pltpu.make_async_copy(t_hbm.at[idx_v[b]], rows_v[b], sem_g[b])

        def r1_copy(c, b):
            return pltpu.make_async_copy(
                r1nb_hbm.at[pl.ds(ebase(c), C)], r1_v[b], sem_r[b])

        def out_copy(c, b):
            return pltpu.make_async_copy(
                out_v[b], s_hbm.at[pl.ds(ebase(c), C)], sem_o[b])

        def compute(b):
            nj = H // LANES
            rows_b, r1_b, out_b = rows_v[b], r1_v[b], out_v[b]

            @plsc.parallel_loop(0, C * nj, unroll=2)
            def pair_body(t):
                e = t // nj
                joff = (t % nj) * LANES
                r1j = r1_b[e, pl.ds(joff, LANES)]
                sh = jnp.zeros((LANES,), jnp.float32)
                sg = jnp.zeros((LANES,), jnp.float32)
                for n in range(NEI):
                    hn = rows_b[e * NEI + n, pl.ds(joff, LANES)]
                    un = rows_b[e * NEI + n, pl.ds(H + joff, LANES)]
                    gh = hn / (1.0 + jnp.exp(r1j - un))
                    sh = sh + hn
                    sg = sg + gh
                out_b[e, pl.ds(joff, LANES)] = sh
                out_b[e, pl.ds(H + joff, LANES)] = sg

        idx_copy(0, 0).start()
        idx_copy(0, 0).wait()
        row_copy(0).start()
        r1_copy(0, 0).start()
        idx_copy(1, 1).start()

        def body(g, carry):
            for half in range(2):
                c = 2 * g + half
                b = half
                row_copy(b).wait()
                r1_copy(c, b).wait()

                @pl.when(c + 2 < NCHUNK)
                def _():
                    idx_copy(c + 2, b).start()

                @pl.when(c + 1 < NCHUNK)
                def _():
                    idx_copy(c + 1, 1 - b).wait()
                    row_copy(1 - b).start()
                    r1_copy(c + 1, 1 - b).start()

                @pl.when(c >= 2)
                def _():
                    out_copy(c, b).wait()

                compute(b)
                out_copy(c, b).start()
            return carry

        lax.fori_loop(0, NCHUNK // 2, body, 0)
        out_copy(NCHUNK - 2, 0).wait()
        out_copy(NCHUNK - 1, 1).wait()

    return sc_gather


def kernel(fmess, bgraph, Wz_w, Wz_b, Wr_w, Ur_w, Ur_b, Wh_w, Wh_b):
    E = fmess.shape[0]
    idx = bgraph.astype(jnp.int32).reshape(-1)

    wzx = Wz_w[:, :H].T
    wzh = Wz_w[:, H:].T
    whx = Wh_w[:, :H].T
    whh = Wh_w[:, H:].T
    wr = Wr_w.T
    ur = Ur_w.T
    bz = Wz_b.reshape(1, H)
    bh = Wh_b.reshape(1, H)
    br = Ur_b.reshape(1, H)

    grid = (E // BT,)
    row_spec = pl.BlockSpec((BT, H), lambda i: (i, 0))
    tab_spec = pl.BlockSpec((BT, 2 * H), lambda i: (i, 0))
    w_spec = pl.BlockSpec((H, H), lambda i: (0, 0))
    b_spec = pl.BlockSpec((1, H), lambda i: (0, 0))

    xwz, xwh, r1nb, T = pl.pallas_call(
        _pre_body,
        grid=grid,
        in_specs=[row_spec, w_spec, b_spec, w_spec, b_spec, w_spec, b_spec, w_spec],
        out_specs=[row_spec, row_spec, row_spec, tab_spec],
        out_shape=[
            jax.ShapeDtypeStruct((E, H), jnp.float32),
            jax.ShapeDtypeStruct((E, H), jnp.float32),
            jax.ShapeDtypeStruct((E, H), jnp.float32),
            jax.ShapeDtypeStruct((E, 2 * H), jnp.float32),
        ],
    )(fmess, wzx, bz, whx, bh, wr, br, ur)

    sc_gather = _make_sc_gather(E, 8)
    upd = pl.pallas_call(
        _upd_body,
        grid=grid,
        in_specs=[tab_spec, row_spec, row_spec, w_spec, w_spec, w_spec],
        out_specs=tab_spec,
        out_shape=jax.ShapeDtypeStruct((E, 2 * H), jnp.float32),
    )

    for _ in range(4):
        S = sc_gather(T, idx, r1nb)
        T = upd(S, xwz, xwh, wzh, whh, ur)

    return T[:, :H]


# speedup vs baseline: 1.0132x; 1.0132x over previous
import functools

import jax
import jax.numpy as jnp
from jax import lax
from jax.experimental import pallas as pl
from jax.experimental.pallas import tpu as pltpu
from jax.experimental.pallas import tpu_sc as plsc

H = 128
NEI = 8
NC, NS = 2, 16
LOG2E = 1.4426950408889634
NW = NC * NS
LANES = 16
BT = 512


def _row_mask0(x, pid, nrows):
    rows = pid * nrows + lax.broadcasted_iota(jnp.int32, x.shape, 0)
    return jnp.where(rows == 0, 0.0, x)


def _pre_body(x_ref, wzx_ref, bz_ref, whx_ref, bh_ref, wr_ref, br_ref, ur_ref,
              r1nb_ref, t_ref):
    pid = pl.program_id(0)
    x = x_ref[...]
    xwz = jnp.dot(x, wzx_ref[...], preferred_element_type=jnp.float32) + bz_ref[...]
    xwh = jnp.dot(x, whx_ref[...], preferred_element_type=jnp.float32) + bh_ref[...]
    r1nb = -(jnp.dot(x, wr_ref[...], preferred_element_type=jnp.float32) + br_ref[...])
    r1nb_ref[...] = r1nb
    h1 = jax.nn.sigmoid(xwz) * jnp.tanh(xwh)
    h1 = _row_mask0(h1, pid, x.shape[0])
    t_ref[:, :H] = h1
    t_ref[:, H:] = jnp.dot(h1, ur_ref[...], preferred_element_type=jnp.float32)


def _upd_body(s_ref, x_ref, wzx_ref, bz_ref, whx_ref, bh_ref, wzh_ref, whh_ref,
              ur_ref, t_ref):
    pid = pl.program_id(0)
    x = x_ref[...]
    sum_h = s_ref[:, :H]
    sum_g = s_ref[:, H:]
    z = jax.nn.sigmoid(
        jnp.dot(x, wzx_ref[...], preferred_element_type=jnp.float32) + bz_ref[...]
        + jnp.dot(sum_h, wzh_ref[...], preferred_element_type=jnp.float32))
    pre = jnp.tanh(
        jnp.dot(x, whx_ref[...], preferred_element_type=jnp.float32) + bh_ref[...]
        + jnp.dot(sum_g, whh_ref[...], preferred_element_type=jnp.float32))
    nh = (1.0 - z) * sum_h + z * pre
    nh = _row_mask0(nh, pid, sum_h.shape[0])
    t_ref[:, :H] = nh
    t_ref[:, H:] = jnp.dot(nh, ur_ref[...], preferred_element_type=jnp.float32)


def _make_sc_gather(E, C):
    EPW = E // NW
    NCHUNK = EPW // C
    mesh = plsc.VectorSubcoreMesh(core_axis_name="c", subcore_axis_name="s",
                                  num_cores=NC, num_subcores=NS)

    assert NCHUNK % 2 == 0

    @functools.partial(
        pl.kernel,
        out_type=jax.ShapeDtypeStruct((E, 2 * H), jnp.float32),
        mesh=mesh,
        scratch_types=[
            [pltpu.VMEM((C * NEI,), jnp.int32)] * 2,
            [pltpu.VMEM((C * NEI, 2 * H), jnp.float32)] * 2,
            [pltpu.VMEM((C, H), jnp.float32)] * 2,
            [pltpu.VMEM((C, 2 * H), jnp.float32)] * 2,
            [pltpu.SemaphoreType.DMA] * 2,
            [pltpu.SemaphoreType.DMA] * 2,
            [pltpu.SemaphoreType.DMA] * 2,
            [pltpu.SemaphoreType.DMA] * 2,
        ],
    )
    def sc_gather(t_hbm, idx_hbm, r1nb_hbm, s_hbm,
                  idx_v, rows_v, r1_v, out_v, sem_i, sem_g, sem_r, sem_o):
        wid = lax.axis_index("s") * NC + lax.axis_index("c")
        base_w = wid * EPW

        def ebase(c):
            return pl.multiple_of(base_w + c * C, 8)

        def ibase(c):
            return pl.multiple_of((base_w + c * C) * NEI, 8)

        def idx_copy(c, b):
            return pltpu.make_async_copy(
                idx_hbm.at[pl.ds(ibase(c), C * NEI)], idx_v[b], sem_i[b])

        def row_copy(b):
            return pltpu.make_async_copy(t_hbm.at[idx_v[b]], rows_v[b], sem_g[b])

        def r1_copy(c, b):
            return pltpu.make_async_copy(
                r1nb_hbm.at[pl.ds(ebase(c), C)], r1_v[b], sem_r[b])

        def out_copy(c, b):
            return pltpu.make_async_copy(
                out_v[b], s_hbm.at[pl.ds(ebase(c), C)], sem_o[b])

        def compute(b):
            nj = H // LANES
            rows_b, r1_b, out_b = rows_v[b], r1_v[b], out_v[b]

            @plsc.parallel_loop(0, C * nj, unroll=2)
            def pair_body(t):
                e = t // nj
                joff = (t % nj) * LANES
                r1j = r1_b[e, pl.ds(joff, LANES)]
                sh = jnp.zeros((LANES,), jnp.float32)
                sg = jnp.zeros((LANES,), jnp.float32)
                for n in range(NEI):
                    hn = rows_b[e * NEI + n, pl.ds(joff, LANES)]
                    un = rows_b[e * NEI + n, pl.ds(H + joff, LANES)]
                    gh = hn / (1.0 + jnp.exp(r1j - un))
                    sh = sh + hn
                    sg = sg + gh
                out_b[e, pl.ds(joff, LANES)] = sh
                out_b[e, pl.ds(H + joff, LANES)] = sg

        idx_copy(0, 0).start()
        idx_copy(0, 0).wait()
        row_copy(0).start()
        r1_copy(0, 0).start()
        idx_copy(1, 1).start()

        def body(g, carry):
            for half in range(2):
                c = 2 * g + half
                b = half
                row_copy(b).wait()
                r1_copy(c, b).wait()

                @pl.when(c + 2 < NCHUNK)
                def _():
                    idx_copy(c + 2, b).start()

                @pl.when(c + 1 < NCHUNK)
                def _():
                    idx_copy(c + 1, 1 - b).wait()
                    row_copy(1 - b).start()
                    r1_copy(c + 1, 1 - b).start()

                @pl.when(c >= 2)
                def _():
                    out_copy(c, b).wait()

                compute(b)
                out_copy(c, b).start()
            return carry

        lax.fori_loop(0, NCHUNK // 2, body, 0)
        out_copy(NCHUNK - 2, 0).wait()
        out_copy(NCHUNK - 1, 1).wait()

    return sc_gather


def kernel(fmess, bgraph, Wz_w, Wz_b, Wr_w, Ur_w, Ur_b, Wh_w, Wh_b):
    E = fmess.shape[0]
    idx = bgraph.astype(jnp.int32).reshape(-1)

    wzx = Wz_w[:, :H].T
    wzh = Wz_w[:, H:].T
    whx = Wh_w[:, :H].T
    whh = Wh_w[:, H:].T
    wr = Wr_w.T
    ur = Ur_w.T
    bz = Wz_b.reshape(1, H)
    bh = Wh_b.reshape(1, H)
    br = Ur_b.reshape(1, H)

    grid = (E // BT,)
    row_spec = pl.BlockSpec((BT, H), lambda i: (i, 0))
    tab_spec = pl.BlockSpec((BT, 2 * H), lambda i: (i, 0))
    w_spec = pl.BlockSpec((H, H), lambda i: (0, 0))
    b_spec = pl.BlockSpec((1, H), lambda i: (0, 0))

    r1nb, T = pl.pallas_call(
        _pre_body,
        grid=grid,
        in_specs=[row_spec, w_spec, b_spec, w_spec, b_spec, w_spec, b_spec, w_spec],
        out_specs=[row_spec, tab_spec],
        out_shape=[
            jax.ShapeDtypeStruct((E, H), jnp.float32),
            jax.ShapeDtypeStruct((E, 2 * H), jnp.float32),
        ],
    )(fmess, wzx, bz, whx, bh, wr, br, ur)

    sc_gather = _make_sc_gather(E, 8)
    upd = pl.pallas_call(
        _upd_body,
        grid=grid,
        in_specs=[tab_spec, row_spec, w_spec, b_spec, w_spec, b_spec, w_spec,
                  w_spec, w_spec],
        out_specs=tab_spec,
        out_shape=jax.ShapeDtypeStruct((E, 2 * H), jnp.float32),
    )

    for _ in range(4):
        S = sc_gather(T, idx, r1nb)
        T = upd(S, fmess, wzx, bz, whx, bh, wzh, whh, ur)

    return T[:, :H]


# speedup vs baseline: 1.0146x; 1.0014x over previous
import functools

import jax
import jax.numpy as jnp
from jax import lax
from jax.experimental import pallas as pl
from jax.experimental.pallas import tpu as pltpu
from jax.experimental.pallas import tpu_sc as plsc

H = 128
NEI = 8
NC, NS = 2, 16
NW = NC * NS
LANES = 16
BT = 512


def _row_mask0(x, pid, nrows):
    rows = pid * nrows + lax.broadcasted_iota(jnp.int32, x.shape, 0)
    return jnp.where(rows == 0, 0.0, x)


def _pre_body(x_ref, wzx_ref, bz_ref, whx_ref, bh_ref, wr_ref, br_ref, ur_ref,
              r1nb_ref, t_ref):
    pid = pl.program_id(0)
    x = x_ref[...]
    xwz = jnp.dot(x, wzx_ref[...], preferred_element_type=jnp.float32) + bz_ref[...]
    xwh = jnp.dot(x, whx_ref[...], preferred_element_type=jnp.float32) + bh_ref[...]
    r1nb = -(jnp.dot(x, wr_ref[...], preferred_element_type=jnp.float32) + br_ref[...])
    r1nb_ref[...] = r1nb
    h1 = jax.nn.sigmoid(xwz) * jnp.tanh(xwh)
    h1 = _row_mask0(h1, pid, x.shape[0])
    t_ref[:, :H] = h1
    t_ref[:, H:] = jnp.dot(h1, ur_ref[...], preferred_element_type=jnp.float32)


def _upd_body(s_ref, x_ref, wzx_ref, bz_ref, whx_ref, bh_ref, wzh_ref, whh_ref,
              ur_ref, t_ref):
    pid = pl.program_id(0)
    x = x_ref[...]
    sum_h = s_ref[:, :H]
    sum_g = s_ref[:, H:]
    z = jax.nn.sigmoid(
        jnp.dot(x, wzx_ref[...], preferred_element_type=jnp.float32) + bz_ref[...]
        + jnp.dot(sum_h, wzh_ref[...], preferred_element_type=jnp.float32))
    pre = jnp.tanh(
        jnp.dot(x, whx_ref[...], preferred_element_type=jnp.float32) + bh_ref[...]
        + jnp.dot(sum_g, whh_ref[...], preferred_element_type=jnp.float32))
    nh = (1.0 - z) * sum_h + z * pre
    nh = _row_mask0(nh, pid, sum_h.shape[0])
    t_ref[:, :H] = nh
    t_ref[:, H:] = jnp.dot(nh, ur_ref[...], preferred_element_type=jnp.float32)


def _make_sc_gather(E, C):
    EPW = E // NW
    NCHUNK = EPW // C
    assert NCHUNK % 2 == 0
    mesh = plsc.VectorSubcoreMesh(core_axis_name="c", subcore_axis_name="s",
                                  num_cores=NC, num_subcores=NS)

    @functools.partial(
        pl.kernel,
        out_type=jax.ShapeDtypeStruct((E, 2 * H), jnp.float32),
        mesh=mesh,
        scratch_types=[
            [pltpu.VMEM((C * NEI,), jnp.int32)] * 2,
            [pltpu.VMEM((C * NEI, 2 * H), jnp.float32)] * 2,
            [pltpu.VMEM((C, H), jnp.float32)] * 2,
            [pltpu.VMEM((C, 2 * H), jnp.float32)] * 2,
            [pltpu.SemaphoreType.DMA] * 2,
            [pltpu.SemaphoreType.DMA] * 2,
            [pltpu.SemaphoreType.DMA] * 2,
            [pltpu.SemaphoreType.DMA] * 2,
        ],
    )
    def sc_gather(t_hbm, idx_hbm, r1nb_hbm, s_hbm,
                  idx_v, rows_v, r1_v, out_v, sem_i, sem_g, sem_r, sem_o):
        wid = lax.axis_index("s") * NC + lax.axis_index("c")
        base_w = wid * EPW

        def ebase(c):
            return pl.multiple_of(base_w + c * C, 8)

        def ibase(c):
            return pl.multiple_of((base_w + c * C) * NEI, 8)

        def idx_copy(c, b):
            return pltpu.make_async_copy(
                idx_hbm.at[pl.ds(ibase(c), C * NEI)], idx_v[b], sem_i[b])

        def row_copy(b):
            return pltpu.make_async_copy(t_hbm.at[idx_v[b]], rows_v[b], sem_g[b])

        def r1_copy(c, b):
            return pltpu.make_async_copy(
                r1nb_hbm.at[pl.ds(ebase(c), C)], r1_v[b], sem_r[b])

        def out_copy(c, b):
            return pltpu.make_async_copy(
                out_v[b], s_hbm.at[pl.ds(ebase(c), C)], sem_o[b])

        def compute(b):
            nj = H // (2 * LANES)
            rows_b, r1_b, out_b = rows_v[b], r1_v[b], out_v[b]

            @plsc.parallel_loop(0, C * nj, unroll=2)
            def pair_body(t):
                e = t // nj
                j = (t % nj) * (2 * LANES)
                r1a = r1_b[e, pl.ds(j, LANES)]
                r1b2 = r1_b[e, pl.ds(j + LANES, LANES)]
                sh0 = jnp.zeros((LANES,), jnp.float32)
                sh1 = jnp.zeros((LANES,), jnp.float32)
                sg0 = jnp.zeros((LANES,), jnp.float32)
                sg1 = jnp.zeros((LANES,), jnp.float32)
                for n in range(NEI):
                    row = e * NEI + n
                    u0 = rows_b[row, pl.ds(H + j, LANES)]
                    u1 = rows_b[row, pl.ds(H + j + LANES, LANES)]
                    ra = 1.0 / (1.0 + jnp.exp(r1a - u0))
                    rb = 1.0 / (1.0 + jnp.exp(r1b2 - u1))
                    h0 = rows_b[row, pl.ds(j, LANES)]
                    h1 = rows_b[row, pl.ds(j + LANES, LANES)]
                    sh0 = sh0 + h0
                    sh1 = sh1 + h1
                    sg0 = sg0 + h0 * ra
                    sg1 = sg1 + h1 * rb
                out_b[e, pl.ds(j, LANES)] = sh0
                out_b[e, pl.ds(j + LANES, LANES)] = sh1
                out_b[e, pl.ds(H + j, LANES)] = sg0
                out_b[e, pl.ds(H + j + LANES, LANES)] = sg1

        idx_copy(0, 0).start()
        idx_copy(0, 0).wait()
        row_copy(0).start()
        r1_copy(0, 0).start()
        idx_copy(1, 1).start()

        def body(g, carry):
            for half in range(2):
                c = 2 * g + half
                b = half
                row_copy(b).wait()
                r1_copy(c, b).wait()

                @pl.when(c + 2 < NCHUNK)
                def _():
                    idx_copy(c + 2, b).start()

                @pl.when(c + 1 < NCHUNK)
                def _():
                    idx_copy(c + 1, 1 - b).wait()
                    row_copy(1 - b).start()
                    r1_copy(c + 1, 1 - b).start()

                @pl.when(c >= 2)
                def _():
                    out_copy(c, b).wait()

                compute(b)
                out_copy(c, b).start()
            return carry

        lax.fori_loop(0, NCHUNK // 2, body, 0)
        out_copy(NCHUNK - 2, 0).wait()
        out_copy(NCHUNK - 1, 1).wait()

    return sc_gather


def kernel(fmess, bgraph, Wz_w, Wz_b, Wr_w, Ur_w, Ur_b, Wh_w, Wh_b):
    E = fmess.shape[0]
    idx = bgraph.astype(jnp.int32).reshape(-1)

    wzx = Wz_w[:, :H].T
    wzh = Wz_w[:, H:].T
    whx = Wh_w[:, :H].T
    whh = Wh_w[:, H:].T
    wr = Wr_w.T
    ur = Ur_w.T
    bz = Wz_b.reshape(1, H)
    bh = Wh_b.reshape(1, H)
    br = Ur_b.reshape(1, H)

    grid = (E // BT,)
    row_spec = pl.BlockSpec((BT, H), lambda i: (i, 0))
    tab_spec = pl.BlockSpec((BT, 2 * H), lambda i: (i, 0))
    w_spec = pl.BlockSpec((H, H), lambda i: (0, 0))
    b_spec = pl.BlockSpec((1, H), lambda i: (0, 0))

    r1nb, T = pl.pallas_call(
        _pre_body,
        grid=grid,
        in_specs=[row_spec, w_spec, b_spec, w_spec, b_spec, w_spec, b_spec, w_spec],
        out_specs=[row_spec, tab_spec],
        out_shape=[
            jax.ShapeDtypeStruct((E, H), jnp.float32),
            jax.ShapeDtypeStruct((E, 2 * H), jnp.float32),
        ],
    )(fmess, wzx, bz, whx, bh, wr, br, ur)

    sc_gather = _make_sc_gather(E, 8)
    upd = pl.pallas_call(
        _upd_body,
        grid=grid,
        in_specs=[tab_spec, row_spec, w_spec, b_spec, w_spec, b_spec, w_spec,
                  w_spec, w_spec],
        out_specs=tab_spec,
        out_shape=jax.ShapeDtypeStruct((E, 2 * H), jnp.float32),
    )

    for _ in range(4):
        S = sc_gather(T, idx, r1nb)
        T = upd(S, fmess, wzx, bz, whx, bh, wzh, whh, ur)

    return T[:, :H]


# speedup vs baseline: 1.2577x; 1.2396x over previous
import functools

import jax
import jax.numpy as jnp
from jax import lax
from jax.experimental import pallas as pl
from jax.experimental.pallas import tpu as pltpu
from jax.experimental.pallas import tpu_sc as plsc

H = 128
NEI = 8
NC, NS = 2, 16
NW = NC * NS
LANES = 16
BT = 512


def _row_mask0(x, pid, nrows):
    rows = pid * nrows + lax.broadcasted_iota(jnp.int32, x.shape, 0)
    return jnp.where(rows == 0, 0.0, x)


def _pre_body(x_ref, wzx_ref, bz_ref, whx_ref, bh_ref, wr_ref, br_ref, ur_ref,
              r1nb_ref, t_ref):
    pid = pl.program_id(0)
    x = x_ref[...]
    xwz = jnp.dot(x, wzx_ref[...], preferred_element_type=jnp.float32) + bz_ref[...]
    xwh = jnp.dot(x, whx_ref[...], preferred_element_type=jnp.float32) + bh_ref[...]
    r1nb = -(jnp.dot(x, wr_ref[...], preferred_element_type=jnp.float32) + br_ref[...])
    r1nb_ref[...] = r1nb
    h1 = jax.nn.sigmoid(xwz) * jnp.tanh(xwh)
    h1 = _row_mask0(h1, pid, x.shape[0])
    t_ref[:, :H] = h1
    t_ref[:, H:] = jnp.dot(h1, ur_ref[...], preferred_element_type=jnp.float32)


def _upd_body(s_ref, x_ref, wzx_ref, bz_ref, whx_ref, bh_ref, wzh_ref, whh_ref,
              ur_ref, t_ref):
    pid = pl.program_id(0)
    x = x_ref[...]
    sum_h = s_ref[:, :H]
    sum_g = s_ref[:, H:]
    z = jax.nn.sigmoid(
        jnp.dot(x, wzx_ref[...], preferred_element_type=jnp.float32) + bz_ref[...]
        + jnp.dot(sum_h, wzh_ref[...], preferred_element_type=jnp.float32))
    pre = jnp.tanh(
        jnp.dot(x, whx_ref[...], preferred_element_type=jnp.float32) + bh_ref[...]
        + jnp.dot(sum_g, whh_ref[...], preferred_element_type=jnp.float32))
    nh = (1.0 - z) * sum_h + z * pre
    nh = _row_mask0(nh, pid, sum_h.shape[0])
    t_ref[:, :H] = nh
    t_ref[:, H:] = jnp.dot(nh, ur_ref[...], preferred_element_type=jnp.float32)


def _upd_final_body(s_ref, x_ref, wzx_ref, bz_ref, whx_ref, bh_ref, wzh_ref,
                    whh_ref, h_ref):
    pid = pl.program_id(0)
    x = x_ref[...]
    sum_h = s_ref[:, :H]
    sum_g = s_ref[:, H:]
    z = jax.nn.sigmoid(
        jnp.dot(x, wzx_ref[...], preferred_element_type=jnp.float32) + bz_ref[...]
        + jnp.dot(sum_h, wzh_ref[...], preferred_element_type=jnp.float32))
    pre = jnp.tanh(
        jnp.dot(x, whx_ref[...], preferred_element_type=jnp.float32) + bh_ref[...]
        + jnp.dot(sum_g, whh_ref[...], preferred_element_type=jnp.float32))
    nh = (1.0 - z) * sum_h + z * pre
    h_ref[...] = _row_mask0(nh, pid, sum_h.shape[0])


def _make_sc_gather(E, C):
    EPW = E // NW
    NCHUNK = EPW // C
    assert NCHUNK % 2 == 1
    mesh = plsc.VectorSubcoreMesh(core_axis_name="c", subcore_axis_name="s",
                                  num_cores=NC, num_subcores=NS)

    @functools.partial(
        pl.kernel,
        out_type=jax.ShapeDtypeStruct((E, 2 * H), jnp.float32),
        mesh=mesh,
        scratch_types=[
            [pltpu.VMEM((C * NEI,), jnp.int32)] * 2,
            [pltpu.VMEM((C * NEI, 2 * H), jnp.float32)] * 2,
            [pltpu.VMEM((C, H), jnp.float32)] * 2,
            [pltpu.VMEM((C, 2 * H), jnp.float32)] * 2,
            [pltpu.SemaphoreType.DMA] * 2,
            [pltpu.SemaphoreType.DMA] * 2,
            [pltpu.SemaphoreType.DMA] * 2,
            [pltpu.SemaphoreType.DMA] * 2,
        ],
    )
    def sc_gather(t_hbm, idx_hbm, r1nb_hbm, s_hbm,
                  idx_v, rows_v, r1_v, out_v, sem_i, sem_g, sem_r, sem_o):
        wid = lax.axis_index("s") * NC + lax.axis_index("c")
        base_w = wid * EPW

        def ebase(c):
            return pl.multiple_of(base_w + c * C, 8)

        def ibase(c):
            return pl.multiple_of((base_w + c * C) * NEI, 8)

        def idx_copy(c, b):
            return pltpu.make_async_copy(
                idx_hbm.at[pl.ds(ibase(c), C * NEI)], idx_v[b], sem_i[b])

        def row_copy(b):
            return pltpu.make_async_copy(t_hbm.at[idx_v[b]], rows_v[b], sem_g[b])

        def r1_copy(c, b):
            return pltpu.make_async_copy(
                r1nb_hbm.at[pl.ds(ebase(c), C)], r1_v[b], sem_r[b])

        def out_copy(c, b):
            return pltpu.make_async_copy(
                out_v[b], s_hbm.at[pl.ds(ebase(c), C)], sem_o[b])

        def compute(b):
            nj = H // (2 * LANES)
            rows_b, r1_b, out_b = rows_v[b], r1_v[b], out_v[b]

            @plsc.parallel_loop(0, C * nj, unroll=2)
            def pair_body(t):
                e = t // nj
                j = (t % nj) * (2 * LANES)
                r1a = r1_b[e, pl.ds(j, LANES)]
                r1b2 = r1_b[e, pl.ds(j + LANES, LANES)]
                sh0 = jnp.zeros((LANES,), jnp.float32)
                sh1 = jnp.zeros((LANES,), jnp.float32)
                sg0 = jnp.zeros((LANES,), jnp.float32)
                sg1 = jnp.zeros((LANES,), jnp.float32)
                for n in range(NEI):
                    row = e * NEI + n
                    u0 = rows_b[row, pl.ds(H + j, LANES)]
                    u1 = rows_b[row, pl.ds(H + j + LANES, LANES)]
                    ra = 1.0 / (1.0 + jnp.exp(r1a - u0))
                    rb = 1.0 / (1.0 + jnp.exp(r1b2 - u1))
                    h0 = rows_b[row, pl.ds(j, LANES)]
                    h1 = rows_b[row, pl.ds(j + LANES, LANES)]
                    sh0 = sh0 + h0
                    sh1 = sh1 + h1
                    sg0 = sg0 + h0 * ra
                    sg1 = sg1 + h1 * rb
                out_b[e, pl.ds(j, LANES)] = sh0
                out_b[e, pl.ds(j + LANES, LANES)] = sh1
                out_b[e, pl.ds(H + j, LANES)] = sg0
                out_b[e, pl.ds(H + j + LANES, LANES)] = sg1

        idx_copy(0, 0).start()
        idx_copy(0, 0).wait()
        row_copy(0).start()
        r1_copy(0, 0).start()
        idx_copy(1, 1).start()
        row_copy(0).wait()
        r1_copy(0, 0).wait()
        idx_copy(2, 0).start()
        idx_copy(1, 1).wait()
        row_copy(1).start()
        r1_copy(1, 1).start()
        compute(0)
        out_copy(0, 0).start()

        def body(g, carry):
            for half in range(2):
                c = 2 * g + 1 + half
                b = 1 - half
                row_copy(b).wait()
                r1_copy(c, b).wait()

                @pl.when(c + 2 < NCHUNK)
                def _():
                    idx_copy(c + 2, b).start()

                @pl.when(c + 1 < NCHUNK)
                def _():
                    idx_copy(c + 1, 1 - b).wait()
                    row_copy(1 - b).start()
                    r1_copy(c + 1, 1 - b).start()

                @pl.when(c >= 2)
                def _():
                    out_copy(c, b).wait()

                compute(b)
                out_copy(c, b).start()
            return carry

        lax.fori_loop(0, (NCHUNK - 1) // 2, body, 0)
        out_copy(NCHUNK - 2, (NCHUNK - 2) % 2).wait()
        out_copy(NCHUNK - 1, (NCHUNK - 1) % 2).wait()

    return sc_gather


def kernel(fmess, bgraph, Wz_w, Wz_b, Wr_w, Ur_w, Ur_b, Wh_w, Wh_b):
    E = fmess.shape[0]
    idx = bgraph.astype(jnp.int32).reshape(-1)

    wzx = Wz_w[:, :H].T
    wzh = Wz_w[:, H:].T
    whx = Wh_w[:, :H].T
    whh = Wh_w[:, H:].T
    wr = Wr_w.T
    ur = Ur_w.T
    bz = Wz_b.reshape(1, H)
    bh = Wh_b.reshape(1, H)
    br = Ur_b.reshape(1, H)

    grid = (E // BT,)
    row_spec = pl.BlockSpec((BT, H), lambda i: (i, 0))
    tab_spec = pl.BlockSpec((BT, 2 * H), lambda i: (i, 0))
    w_spec = pl.BlockSpec((H, H), lambda i: (0, 0))
    b_spec = pl.BlockSpec((1, H), lambda i: (0, 0))

    r1nb, T = pl.pallas_call(
        _pre_body,
        grid=grid,
        in_specs=[row_spec, w_spec, b_spec, w_spec, b_spec, w_spec, b_spec, w_spec],
        out_specs=[row_spec, tab_spec],
        out_shape=[
            jax.ShapeDtypeStruct((E, H), jnp.float32),
            jax.ShapeDtypeStruct((E, 2 * H), jnp.float32),
        ],
    )(fmess, wzx, bz, whx, bh, wr, br, ur)

    sc_gather = _make_sc_gather(E, 16)
    upd = pl.pallas_call(
        _upd_body,
        grid=grid,
        in_specs=[tab_spec, row_spec, w_spec, b_spec, w_spec, b_spec, w_spec,
                  w_spec, w_spec],
        out_specs=tab_spec,
        out_shape=jax.ShapeDtypeStruct((E, 2 * H), jnp.float32),
    )

    upd_final = pl.pallas_call(
        _upd_final_body,
        grid=grid,
        in_specs=[tab_spec, row_spec, w_spec, b_spec, w_spec, b_spec, w_spec,
                  w_spec],
        out_specs=row_spec,
        out_shape=jax.ShapeDtypeStruct((E, H), jnp.float32),
    )

    for _ in range(3):
        S = sc_gather(T, idx, r1nb)
        T = upd(S, fmess, wzx, bz, whx, bh, wzh, whh, ur)
    S = sc_gather(T, idx, r1nb)
    return upd_final(S, fmess, wzx, bz, whx, bh, wzh, whh)


# speedup vs baseline: 1.5417x; 1.2258x over previous
import functools

import jax
import jax.numpy as jnp
from jax import lax
from jax.experimental import pallas as pl
from jax.experimental.pallas import tpu as pltpu
from jax.experimental.pallas import tpu_sc as plsc

H = 128
NEI = 8
NC, NS = 2, 16
NW = NC * NS
LANES = 16
BT = 2560


def _row_mask0(x, pid, nrows):
    rows = pid * nrows + lax.broadcasted_iota(jnp.int32, x.shape, 0)
    return jnp.where(rows == 0, 0.0, x)


def _pre_body(x_ref, wzx_ref, bz_ref, whx_ref, bh_ref, wr_ref, br_ref, ur_ref,
              r1nb_ref, t_ref):
    pid = pl.program_id(0)
    x = x_ref[...]
    xwz = jnp.dot(x, wzx_ref[...], preferred_element_type=jnp.float32) + bz_ref[...]
    xwh = jnp.dot(x, whx_ref[...], preferred_element_type=jnp.float32) + bh_ref[...]
    r1nb = -(jnp.dot(x, wr_ref[...], preferred_element_type=jnp.float32) + br_ref[...])
    r1nb_ref[...] = r1nb
    h1 = jax.nn.sigmoid(xwz) * jnp.tanh(xwh)
    h1 = _row_mask0(h1, pid, x.shape[0])
    t_ref[:, :H] = h1
    t_ref[:, H:] = jnp.dot(h1, ur_ref[...], preferred_element_type=jnp.float32)


def _upd_body(s_ref, x_ref, wzx_ref, bz_ref, whx_ref, bh_ref, wzh_ref, whh_ref,
              ur_ref, t_ref):
    pid = pl.program_id(0)
    x = x_ref[...]
    sum_h = s_ref[:, :H]
    sum_g = s_ref[:, H:]
    z = jax.nn.sigmoid(
        jnp.dot(x, wzx_ref[...], preferred_element_type=jnp.float32) + bz_ref[...]
        + jnp.dot(sum_h, wzh_ref[...], preferred_element_type=jnp.float32))
    pre = jnp.tanh(
        jnp.dot(x, whx_ref[...], preferred_element_type=jnp.float32) + bh_ref[...]
        + jnp.dot(sum_g, whh_ref[...], preferred_element_type=jnp.float32))
    nh = (1.0 - z) * sum_h + z * pre
    nh = _row_mask0(nh, pid, sum_h.shape[0])
    t_ref[:, :H] = nh
    t_ref[:, H:] = jnp.dot(nh, ur_ref[...], preferred_element_type=jnp.float32)


def _upd_final_body(s_ref, x_ref, wzx_ref, bz_ref, whx_ref, bh_ref, wzh_ref,
                    whh_ref, h_ref):
    pid = pl.program_id(0)
    x = x_ref[...]
    sum_h = s_ref[:, :H]
    sum_g = s_ref[:, H:]
    z = jax.nn.sigmoid(
        jnp.dot(x, wzx_ref[...], preferred_element_type=jnp.float32) + bz_ref[...]
        + jnp.dot(sum_h, wzh_ref[...], preferred_element_type=jnp.float32))
    pre = jnp.tanh(
        jnp.dot(x, whx_ref[...], preferred_element_type=jnp.float32) + bh_ref[...]
        + jnp.dot(sum_g, whh_ref[...], preferred_element_type=jnp.float32))
    nh = (1.0 - z) * sum_h + z * pre
    h_ref[...] = _row_mask0(nh, pid, sum_h.shape[0])


def _make_sc_gather(E, C):
    EPW = E // NW
    NCHUNK = EPW // C
    assert NCHUNK % 2 == 1
    mesh = plsc.VectorSubcoreMesh(core_axis_name="c", subcore_axis_name="s",
                                  num_cores=NC, num_subcores=NS)

    @functools.partial(
        pl.kernel,
        out_type=jax.ShapeDtypeStruct((E, 2 * H), jnp.float32),
        mesh=mesh,
        scratch_types=[
            [pltpu.VMEM((C * NEI,), jnp.int32)] * 2,
            [pltpu.VMEM((C * NEI, 2 * H), jnp.float32)] * 2,
            [pltpu.VMEM((C, H), jnp.float32)] * 2,
            [pltpu.VMEM((C, 2 * H), jnp.float32)] * 2,
            [pltpu.SemaphoreType.DMA] * 2,
            [pltpu.SemaphoreType.DMA] * 2,
            [pltpu.SemaphoreType.DMA] * 2,
            [pltpu.SemaphoreType.DMA] * 2,
        ],
    )
    def sc_gather(t_hbm, idx_hbm, r1nb_hbm, s_hbm,
                  idx_v, rows_v, r1_v, out_v, sem_i, sem_g, sem_r, sem_o):
        wid = lax.axis_index("s") * NC + lax.axis_index("c")
        base_w = wid * EPW

        def ebase(c):
            return pl.multiple_of(base_w + c * C, 8)

        def ibase(c):
            return pl.multiple_of((base_w + c * C) * NEI, 8)

        def idx_copy(c, b):
            return pltpu.make_async_copy(
                idx_hbm.at[pl.ds(ibase(c), C * NEI)], idx_v[b], sem_i[b])

        def row_copy(b):
            return pltpu.make_async_copy(t_hbm.at[idx_v[b]], rows_v[b], sem_g[b])

        def r1_copy(c, b):
            return pltpu.make_async_copy(
                r1nb_hbm.at[pl.ds(ebase(c), C)], r1_v[b], sem_r[b])

        def out_copy(c, b):
            return pltpu.make_async_copy(
                out_v[b], s_hbm.at[pl.ds(ebase(c), C)], sem_o[b])

        def compute(b):
            nj = H // (2 * LANES)
            rows_b, r1_b, out_b = rows_v[b], r1_v[b], out_v[b]

            @plsc.parallel_loop(0, C * nj, unroll=2)
            def pair_body(t):
                e = t // nj
                j = (t % nj) * (2 * LANES)
                r1a = r1_b[e, pl.ds(j, LANES)]
                r1b2 = r1_b[e, pl.ds(j + LANES, LANES)]
                sh0 = jnp.zeros((LANES,), jnp.float32)
                sh1 = jnp.zeros((LANES,), jnp.float32)
                sg0 = jnp.zeros((LANES,), jnp.float32)
                sg1 = jnp.zeros((LANES,), jnp.float32)
                for n in range(NEI):
                    row = e * NEI + n
                    u0 = rows_b[row, pl.ds(H + j, LANES)]
                    u1 = rows_b[row, pl.ds(H + j + LANES, LANES)]
                    ra = 1.0 / (1.0 + jnp.exp(r1a - u0))
                    rb = 1.0 / (1.0 + jnp.exp(r1b2 - u1))
                    h0 = rows_b[row, pl.ds(j, LANES)]
                    h1 = rows_b[row, pl.ds(j + LANES, LANES)]
                    sh0 = sh0 + h0
                    sh1 = sh1 + h1
                    sg0 = sg0 + h0 * ra
                    sg1 = sg1 + h1 * rb
                out_b[e, pl.ds(j, LANES)] = sh0
                out_b[e, pl.ds(j + LANES, LANES)] = sh1
                out_b[e, pl.ds(H + j, LANES)] = sg0
                out_b[e, pl.ds(H + j + LANES, LANES)] = sg1

        idx_copy(0, 0).start()
        idx_copy(0, 0).wait()
        row_copy(0).start()
        r1_copy(0, 0).start()
        idx_copy(1, 1).start()
        row_copy(0).wait()
        r1_copy(0, 0).wait()
        idx_copy(2, 0).start()
        idx_copy(1, 1).wait()
        row_copy(1).start()
        r1_copy(1, 1).start()
        compute(0)
        out_copy(0, 0).start()

        def body(g, carry):
            for half in range(2):
                c = 2 * g + 1 + half
                b = 1 - half
                row_copy(b).wait()
                r1_copy(c, b).wait()

                @pl.when(c + 2 < NCHUNK)
                def _():
                    idx_copy(c + 2, b).start()

                @pl.when(c + 1 < NCHUNK)
                def _():
                    idx_copy(c + 1, 1 - b).wait()
                    row_copy(1 - b).start()
                    r1_copy(c + 1, 1 - b).start()

                @pl.when(c >= 2)
                def _():
                    out_copy(c, b).wait()

                compute(b)
                out_copy(c, b).start()
            return carry

        lax.fori_loop(0, (NCHUNK - 1) // 2, body, 0)
        out_copy(NCHUNK - 2, (NCHUNK - 2) % 2).wait()
        out_copy(NCHUNK - 1, (NCHUNK - 1) % 2).wait()

    return sc_gather


def kernel(fmess, bgraph, Wz_w, Wz_b, Wr_w, Ur_w, Ur_b, Wh_w, Wh_b):
    E = fmess.shape[0]
    idx = bgraph.astype(jnp.int32).reshape(-1)

    wzx = Wz_w[:, :H].T
    wzh = Wz_w[:, H:].T
    whx = Wh_w[:, :H].T
    whh = Wh_w[:, H:].T
    wr = Wr_w.T
    ur = Ur_w.T
    bz = Wz_b.reshape(1, H)
    bh = Wh_b.reshape(1, H)
    br = Ur_b.reshape(1, H)

    grid = (E // BT,)
    row_spec = pl.BlockSpec((BT, H), lambda i: (i, 0))
    tab_spec = pl.BlockSpec((BT, 2 * H), lambda i: (i, 0))
    w_spec = pl.BlockSpec((H, H), lambda i: (0, 0))
    b_spec = pl.BlockSpec((1, H), lambda i: (0, 0))

    r1nb, T = pl.pallas_call(
        _pre_body,
        grid=grid,
        in_specs=[row_spec, w_spec, b_spec, w_spec, b_spec, w_spec, b_spec, w_spec],
        out_specs=[row_spec, tab_spec],
        out_shape=[
            jax.ShapeDtypeStruct((E, H), jnp.float32),
            jax.ShapeDtypeStruct((E, 2 * H), jnp.float32),
        ],
    )(fmess, wzx, bz, whx, bh, wr, br, ur)

    sc_gather = _make_sc_gather(E, 16)
    upd = pl.pallas_call(
        _upd_body,
        grid=grid,
        in_specs=[tab_spec, row_spec, w_spec, b_spec, w_spec, b_spec, w_spec,
                  w_spec, w_spec],
        out_specs=tab_spec,
        out_shape=jax.ShapeDtypeStruct((E, 2 * H), jnp.float32),
    )

    upd_final = pl.pallas_call(
        _upd_final_body,
        grid=grid,
        in_specs=[tab_spec, row_spec, w_spec, b_spec, w_spec, b_spec, w_spec,
                  w_spec],
        out_specs=row_spec,
        out_shape=jax.ShapeDtypeStruct((E, H), jnp.float32),
    )

    for _ in range(3):
        S = sc_gather(T, idx, r1nb)
        T = upd(S, fmess, wzx, bz, whx, bh, wzh, whh, ur)
    S = sc_gather(T, idx, r1nb)
    return upd_final(S, fmess, wzx, bz, whx, bh, wzh, whh)


# speedup vs baseline: 1.5836x; 1.0272x over previous
import functools

import jax
import jax.numpy as jnp
from jax import lax
from jax.experimental import pallas as pl
from jax.experimental.pallas import tpu as pltpu
from jax.experimental.pallas import tpu_sc as plsc

H = 128
NEI = 8
NC, NS = 2, 16
NW = NC * NS
LANES = 16
BT = 6400


def _row_mask0(x, pid, nrows):
    rows = pid * nrows + lax.broadcasted_iota(jnp.int32, x.shape, 0)
    return jnp.where(rows == 0, 0.0, x)


def _pre_body(x_ref, wzx_ref, bz_ref, whx_ref, bh_ref, wr_ref, br_ref, ur_ref,
              r1nb_ref, t_ref):
    pid = pl.program_id(0)
    x = x_ref[...]
    xwz = jnp.dot(x, wzx_ref[...], preferred_element_type=jnp.float32) + bz_ref[...]
    xwh = jnp.dot(x, whx_ref[...], preferred_element_type=jnp.float32) + bh_ref[...]
    r1nb = -(jnp.dot(x, wr_ref[...], preferred_element_type=jnp.float32) + br_ref[...])
    r1nb_ref[...] = r1nb
    h1 = jax.nn.sigmoid(xwz) * jnp.tanh(xwh)
    h1 = _row_mask0(h1, pid, x.shape[0])
    t_ref[:, :H] = h1
    t_ref[:, H:] = jnp.dot(h1, ur_ref[...], preferred_element_type=jnp.float32)


def _upd_body(s_ref, x_ref, wzx_ref, bz_ref, whx_ref, bh_ref, wzh_ref, whh_ref,
              ur_ref, t_ref):
    pid = pl.program_id(0)
    x = x_ref[...]
    sum_h = s_ref[:, :H]
    sum_g = s_ref[:, H:]
    z = jax.nn.sigmoid(
        jnp.dot(x, wzx_ref[...], preferred_element_type=jnp.float32) + bz_ref[...]
        + jnp.dot(sum_h, wzh_ref[...], preferred_element_type=jnp.float32))
    pre = jnp.tanh(
        jnp.dot(x, whx_ref[...], preferred_element_type=jnp.float32) + bh_ref[...]
        + jnp.dot(sum_g, whh_ref[...], preferred_element_type=jnp.float32))
    nh = (1.0 - z) * sum_h + z * pre
    nh = _row_mask0(nh, pid, sum_h.shape[0])
    t_ref[:, :H] = nh
    t_ref[:, H:] = jnp.dot(nh, ur_ref[...], preferred_element_type=jnp.float32)


def _upd_final_body(s_ref, x_ref, wzx_ref, bz_ref, whx_ref, bh_ref, wzh_ref,
                    whh_ref, h_ref):
    pid = pl.program_id(0)
    x = x_ref[...]
    sum_h = s_ref[:, :H]
    sum_g = s_ref[:, H:]
    z = jax.nn.sigmoid(
        jnp.dot(x, wzx_ref[...], preferred_element_type=jnp.float32) + bz_ref[...]
        + jnp.dot(sum_h, wzh_ref[...], preferred_element_type=jnp.float32))
    pre = jnp.tanh(
        jnp.dot(x, whx_ref[...], preferred_element_type=jnp.float32) + bh_ref[...]
        + jnp.dot(sum_g, whh_ref[...], preferred_element_type=jnp.float32))
    nh = (1.0 - z) * sum_h + z * pre
    h_ref[...] = _row_mask0(nh, pid, sum_h.shape[0])


def _make_sc_gather(E, C):
    EPW = E // NW
    NCHUNK = EPW // C
    assert NCHUNK % 2 == 1
    mesh = plsc.VectorSubcoreMesh(core_axis_name="c", subcore_axis_name="s",
                                  num_cores=NC, num_subcores=NS)

    @functools.partial(
        pl.kernel,
        out_type=jax.ShapeDtypeStruct((E, 2 * H), jnp.float32),
        mesh=mesh,
        scratch_types=[
            [pltpu.VMEM((C * NEI,), jnp.int32)] * 2,
            [pltpu.VMEM((C * NEI, 2 * H), jnp.float32)] * 2,
            [pltpu.VMEM((C, H), jnp.float32)] * 2,
            [pltpu.VMEM((C, 2 * H), jnp.float32)] * 2,
            [pltpu.SemaphoreType.DMA] * 2,
            [pltpu.SemaphoreType.DMA] * 2,
            [pltpu.SemaphoreType.DMA] * 2,
            [pltpu.SemaphoreType.DMA] * 2,
        ],
    )
    def sc_gather(t_hbm, idx_hbm, r1nb_hbm, s_hbm,
                  idx_v, rows_v, r1_v, out_v, sem_i, sem_g, sem_r, sem_o):
        wid = lax.axis_index("s") * NC + lax.axis_index("c")
        base_w = wid * EPW

        def ebase(c):
            return pl.multiple_of(base_w + c * C, 8)

        def ibase(c):
            return pl.multiple_of((base_w + c * C) * NEI, 8)

        def idx_copy(c, b):
            return pltpu.make_async_copy(
                idx_hbm.at[pl.ds(ibase(c), C * NEI)], idx_v[b], sem_i[b])

        def row_copy(b):
            return pltpu.make_async_copy(t_hbm.at[idx_v[b]], rows_v[b], sem_g[b])

        def r1_copy(c, b):
            return pltpu.make_async_copy(
                r1nb_hbm.at[pl.ds(ebase(c), C)], r1_v[b], sem_r[b])

        def out_copy(c, b):
            return pltpu.make_async_copy(
                out_v[b], s_hbm.at[pl.ds(ebase(c), C)], sem_o[b])

        def compute(b):
            nj = H // (2 * LANES)
            rows_b, r1_b, out_b = rows_v[b], r1_v[b], out_v[b]

            @plsc.parallel_loop(0, C * nj, unroll=2)
            def pair_body(t):
                e = t // nj
                j = (t % nj) * (2 * LANES)
                r1a = r1_b[e, pl.ds(j, LANES)]
                r1b2 = r1_b[e, pl.ds(j + LANES, LANES)]
                sh0 = jnp.zeros((LANES,), jnp.float32)
                sh1 = jnp.zeros((LANES,), jnp.float32)
                sg0 = jnp.zeros((LANES,), jnp.float32)
                sg1 = jnp.zeros((LANES,), jnp.float32)
                for n in range(NEI):
                    row = e * NEI + n
                    u0 = rows_b[row, pl.ds(H + j, LANES)]
                    u1 = rows_b[row, pl.ds(H + j + LANES, LANES)]
                    ra = 1.0 / (1.0 + jnp.exp(r1a - u0))
                    rb = 1.0 / (1.0 + jnp.exp(r1b2 - u1))
                    h0 = rows_b[row, pl.ds(j, LANES)]
                    h1 = rows_b[row, pl.ds(j + LANES, LANES)]
                    sh0 = sh0 + h0
                    sh1 = sh1 + h1
                    sg0 = sg0 + h0 * ra
                    sg1 = sg1 + h1 * rb
                out_b[e, pl.ds(j, LANES)] = sh0
                out_b[e, pl.ds(j + LANES, LANES)] = sh1
                out_b[e, pl.ds(H + j, LANES)] = sg0
                out_b[e, pl.ds(H + j + LANES, LANES)] = sg1

        idx_copy(0, 0).start()
        idx_copy(0, 0).wait()
        row_copy(0).start()
        r1_copy(0, 0).start()
        idx_copy(1, 1).start()
        row_copy(0).wait()
        r1_copy(0, 0).wait()
        idx_copy(2, 0).start()
        idx_copy(1, 1).wait()
        row_copy(1).start()
        r1_copy(1, 1).start()
        compute(0)
        out_copy(0, 0).start()

        def body(g, carry):
            for half in range(2):
                c = 2 * g + 1 + half
                b = 1 - half
                row_copy(b).wait()
                r1_copy(c, b).wait()

                @pl.when(c + 2 < NCHUNK)
                def _():
                    idx_copy(c + 2, b).start()

                @pl.when(c + 1 < NCHUNK)
                def _():
                    idx_copy(c + 1, 1 - b).wait()
                    row_copy(1 - b).start()
                    r1_copy(c + 1, 1 - b).start()

                @pl.when(c >= 2)
                def _():
                    out_copy(c, b).wait()

                compute(b)
                out_copy(c, b).start()
            return carry

        lax.fori_loop(0, (NCHUNK - 1) // 2, body, 0)
        out_copy(NCHUNK - 2, (NCHUNK - 2) % 2).wait()
        out_copy(NCHUNK - 1, (NCHUNK - 1) % 2).wait()

    return sc_gather


def kernel(fmess, bgraph, Wz_w, Wz_b, Wr_w, Ur_w, Ur_b, Wh_w, Wh_b):
    E = fmess.shape[0]
    idx = bgraph.astype(jnp.int32).reshape(-1)

    wzx = Wz_w[:, :H].T
    wzh = Wz_w[:, H:].T
    whx = Wh_w[:, :H].T
    whh = Wh_w[:, H:].T
    wr = Wr_w.T
    ur = Ur_w.T
    bz = Wz_b.reshape(1, H)
    bh = Wh_b.reshape(1, H)
    br = Ur_b.reshape(1, H)

    grid = (E // BT,)
    row_spec = pl.BlockSpec((BT, H), lambda i: (i, 0))
    tab_spec = pl.BlockSpec((BT, 2 * H), lambda i: (i, 0))
    w_spec = pl.BlockSpec((H, H), lambda i: (0, 0))
    b_spec = pl.BlockSpec((1, H), lambda i: (0, 0))

    r1nb, T = pl.pallas_call(
        _pre_body,
        grid=grid,
        in_specs=[row_spec, w_spec, b_spec, w_spec, b_spec, w_spec, b_spec, w_spec],
        out_specs=[row_spec, tab_spec],
        out_shape=[
            jax.ShapeDtypeStruct((E, H), jnp.float32),
            jax.ShapeDtypeStruct((E, 2 * H), jnp.float32),
        ],
    )(fmess, wzx, bz, whx, bh, wr, br, ur)

    sc_gather = _make_sc_gather(E, 16)
    upd = pl.pallas_call(
        _upd_body,
        grid=grid,
        in_specs=[tab_spec, row_spec, w_spec, b_spec, w_spec, b_spec, w_spec,
                  w_spec, w_spec],
        out_specs=tab_spec,
        out_shape=jax.ShapeDtypeStruct((E, 2 * H), jnp.float32),
    )

    upd_final = pl.pallas_call(
        _upd_final_body,
        grid=grid,
        in_specs=[tab_spec, row_spec, w_spec, b_spec, w_spec, b_spec, w_spec,
                  w_spec],
        out_specs=row_spec,
        out_shape=jax.ShapeDtypeStruct((E, H), jnp.float32),
    )

    for _ in range(3):
        S = sc_gather(T, idx, r1nb)
        T = upd(S, fmess, wzx, bz, whx, bh, wzh, whh, ur)
    S = sc_gather(T, idx, r1nb)
    return upd_final(S, fmess, wzx, bz, whx, bh, wzh, whh)
